# baseline (speedup 1.0000x reference)
import functools
import math

import jax
import jax.numpy as jnp
from jax import lax
from jax.experimental import pallas as pl
from jax.experimental.pallas import tpu as pltpu
from jax.experimental.pallas import tpu_sc as plsc

VOCAB = 1000000
DIM = 64
MAX_LEN = 200
BATCH = 1024
SEQ = 200

NC = 2
NS = 16
NW = NC * NS

ROWS = BATCH * SEQ
ROWS_PER_W = ROWS // NW
CHUNK = 64
NCHUNK = ROWS_PER_W // CHUNK
NB = 5
NO = 2
PAIRS = ROWS // 2
CPAIR = CHUNK // 2
PE_ROWS = MAX_LEN // 2 + CPAIR
LANES = 16
SCALE = math.sqrt(DIM)


def _splat(vec, lane):
    idx = jnp.full((LANES,), lane, dtype=jnp.int32)
    return lax.gather(
        vec,
        idx[:, None],
        dimension_numbers=lax.GatherDimensionNumbers(
            offset_dims=(), collapsed_slice_dims=(0,), start_index_map=(0,)),
        slice_sizes=(1,),
        mode=lax.GatherScatterMode.PROMISE_IN_BOUNDS)


TBLK = 512


NBLK = (VOCAB + TBLK - 1) // TBLK
PAIRS_T2 = NBLK * (TBLK // 2)


def _fmt_body(tt_ref, t2_ref):
    t = tt_ref[...].T
    lo = jnp.concatenate([t[0:128], t[256:384]], axis=0)
    hi = jnp.concatenate([t[128:256], t[384:512]], axis=0)
    t2_ref[...] = jnp.concatenate([lo, hi], axis=1)


_fmt = pl.pallas_call(
    _fmt_body,
    grid=(NBLK,),
    in_specs=[pl.BlockSpec((DIM, TBLK), lambda i: (0, i))],
    out_specs=pl.BlockSpec((TBLK // 2, 128), lambda i: (i, 0)),
    out_shape=jax.ShapeDtypeStruct((PAIRS_T2, 128), jnp.float32),
)


def _make_pe():
    w = jnp.exp(-jnp.arange(0, DIM, 2, dtype=jnp.float32) * math.log(10000.0) / DIM)
    p = jnp.arange(0, MAX_LEN, dtype=jnp.float32).reshape(MAX_LEN, 1)
    pe = jnp.zeros((MAX_LEN, DIM), dtype=jnp.float32)
    pe = pe.at[:, 0::2].set(jnp.sin(p * w))
    pe = pe.at[:, 1::2].set(jnp.cos(p * w))
    return pe


@functools.partial(
    pl.kernel,
    mesh=plsc.VectorSubcoreMesh(core_axis_name="c", subcore_axis_name="s"),
    out_type=jax.ShapeDtypeStruct((PAIRS, 128), jnp.float32),
    scratch_types=[
        pltpu.VMEM((PE_ROWS, 128), jnp.float32),
        pltpu.VMEM((NB, CHUNK), jnp.int32),
        pltpu.VMEM((NB, CHUNK), jnp.int32),
        pltpu.VMEM((NB, CHUNK, 128), jnp.float32),
        pltpu.VMEM((NO, CPAIR, 128), jnp.float32),
        pltpu.SemaphoreType.DMA((NB,)),
        pltpu.SemaphoreType.DMA((NO,)),
    ],
    compiler_params=pltpu.CompilerParams(use_tc_tiling_on_sc=True),
)
def _sc_embed(idx_hbm, pe_hbm, t2_hbm, out_hbm, pe_v, iv, pv, gbuf, obuf,
              gsem, osem):
    wid = lax.axis_index("s") * NC + lax.axis_index("c")
    base = wid * ROWS_PER_W
    qbase = wid * (ROWS_PER_W // 2)
    pltpu.sync_copy(pe_hbm, pe_v)

    def stage_and_gather(c, b):
        off = base + c * CHUNK
        pltpu.sync_copy(idx_hbm.at[pl.ds(off, CHUNK)], iv.at[b])
        for t in range(CHUNK // LANES):
            sl = pl.ds(t * LANES, LANES)
            v16 = iv[b, sl]
            pv[b, sl] = (
                jax.lax.shift_left(
                    jax.lax.shift_right_logical(v16, 8), 7)
                + (v16 & 127))
        pltpu.async_copy(t2_hbm.at[pv.at[b]], gbuf.at[b], gsem.at[b])

    for b in range(NB):
        stage_and_gather(b, b)

    def group_body(g, carry):
        for b in range(NB):
            c = g * NB + b
            o = b % NO
            @pl.when(c >= NO)
            def _():
                pltpu.make_async_copy(
                    obuf.at[o], out_hbm.at[pl.ds(0, CPAIR)], osem.at[o]).wait()

            psrc = lax.rem(c * CPAIR, MAX_LEN // 2)
            pltpu.make_async_copy(
                t2_hbm.at[pv.at[b]], gbuf.at[b], gsem.at[b]).wait()

            def grp_body(t, tcarry):
                hv = jax.lax.shift_right_logical(
                    iv[b, pl.ds(t * LANES, LANES)], 7) & 1
                for r in range(LANES):
                    j = t * LANES + r
                    k = t * (LANES // 2) + r // 2
                    par = r & 1
                    m = jnp.int32(0) - _splat(hv, r)
                    nm = ~m
                    for v in range(DIM // LANES):
                        g0 = gbuf[b, j, pl.ds(v * LANES, LANES)]
                        g1 = gbuf[b, j, pl.ds(DIM + v * LANES, LANES)]
                        g0i = lax.bitcast_convert_type(g0, jnp.int32)
                        g1i = lax.bitcast_convert_type(g1, jnp.int32)
                        g16 = lax.bitcast_convert_type(
                            (g0i & nm) | (g1i & m), jnp.float32)
                        p16 = pe_v[psrc + k, pl.ds(par * DIM + v * LANES, LANES)]
                        obuf[o, k, pl.ds(par * DIM + v * LANES, LANES)] = (
                            g16 * SCALE + p16)
                return tcarry

            lax.fori_loop(0, CHUNK // LANES, grp_body, 0)
            pltpu.async_copy(
                obuf.at[o], out_hbm.at[pl.ds(qbase + c * CPAIR, CPAIR)],
                osem.at[o])

            @pl.when(g < NCHUNK // NB - 1)
            def _():
                stage_and_gather(c + NB, b)
        return carry

    lax.fori_loop(0, NCHUNK // NB, group_body, 0)

    for o in range(NO):
        pltpu.make_async_copy(
            obuf.at[o], out_hbm.at[pl.ds(0, CPAIR)], osem.at[o]).wait()


def kernel(x, table):
    pe2 = _make_pe().reshape(MAX_LEN // 2, 128)
    pe2rep = jnp.concatenate([pe2, pe2[:CPAIR]], axis=0)
    t2 = _fmt(jnp.swapaxes(table, 0, 1))
    idx = x.reshape(-1).astype(jnp.int32)
    out = _sc_embed(idx, pe2rep, t2)
    return out.reshape(BATCH, SEQ, DIM)


# speedup vs baseline: 1.0135x; 1.0135x over previous
import functools
import math

import jax
import jax.numpy as jnp
from jax import lax
from jax.experimental import pallas as pl
from jax.experimental.pallas import tpu as pltpu
from jax.experimental.pallas import tpu_sc as plsc

VOCAB = 1000000
DIM = 64
MAX_LEN = 200
BATCH = 1024
SEQ = 200

NC = 2
NS = 16
NW = NC * NS

ROWS = BATCH * SEQ
ROWS_PER_W = ROWS // NW
CHUNK = 80
NCHUNK = ROWS_PER_W // CHUNK
PE_ROWS = MAX_LEN + CHUNK
NG = 8
NO = 4
LANES = 16
SCALE = math.sqrt(DIM)

TBLK = 512
NBLK = (VOCAB + TBLK - 1) // TBLK
TROWS = NBLK * TBLK

def _fmt_body(tt_ref, t2_ref):
    blk = tt_ref[...]
    eye = jnp.eye(DIM, dtype=jnp.float32)
    t = lax.dot_general(blk, eye, (((0,), (0,)), ((), ())),
                        preferred_element_type=jnp.float32)
    lo = jnp.concatenate([t[0:128], t[256:384]], axis=0)
    hi = jnp.concatenate([t[128:256], t[384:512]], axis=0)
    t2_ref[...] = jnp.concatenate([lo, hi], axis=1)


_fmt = pl.pallas_call(
    _fmt_body,
    grid=(NBLK,),
    in_specs=[pl.BlockSpec((DIM, TBLK), lambda i: (0, i))],
    out_specs=pl.BlockSpec((TBLK // 2, 128), lambda i: (i, 0)),
    out_shape=jax.ShapeDtypeStruct((TROWS // 2, 128), jnp.float32),
)


def _make_pe():
    w = jnp.exp(-jnp.arange(0, DIM, 2, dtype=jnp.float32) * math.log(10000.0) / DIM)
    p = jnp.arange(0, MAX_LEN, dtype=jnp.float32).reshape(MAX_LEN, 1)
    pe = jnp.zeros((MAX_LEN, DIM), dtype=jnp.float32)
    pe = pe.at[:, 0::2].set(jnp.sin(p * w))
    pe = pe.at[:, 1::2].set(jnp.cos(p * w))
    return pe


@functools.partial(
    pl.kernel,
    mesh=plsc.VectorSubcoreMesh(core_axis_name="c", subcore_axis_name="s"),
    out_type=jax.ShapeDtypeStruct((ROWS, DIM), jnp.float32),
    scratch_types=[
        pltpu.VMEM((NCHUNK, CHUNK), jnp.int32),
        pltpu.VMEM((NCHUNK, CHUNK), jnp.int32),
        pltpu.VMEM((PE_ROWS, DIM), jnp.float32),
        pltpu.VMEM((NG, CHUNK, DIM), jnp.float32),
        pltpu.VMEM((NO, CHUNK, DIM), jnp.float32),
        pltpu.SemaphoreType.DMA((NG,)),
        pltpu.SemaphoreType.DMA((NO,)),
    ],
    compiler_params=pltpu.CompilerParams(use_tc_tiling_on_sc=False),
)
def _sc_embed(idx_hbm, pe_hbm, t2_hbm, out_hbm, iv, pv, pe_v, gbuf, obuf,
              gsem, osem):
    wid = lax.axis_index("s") * NC + lax.axis_index("c")
    base = wid * ROWS_PER_W
    pltpu.sync_copy(idx_hbm.at[pl.ds(wid * NCHUNK, NCHUNK)], iv)
    pltpu.sync_copy(pe_hbm, pe_v)
    def pv_body(c, pcarry):
        for t in range(CHUNK // LANES):
            sl = pl.ds(t * LANES, LANES)
            v16 = iv[c, sl]
            pv[c, sl] = (
                jax.lax.shift_left(jax.lax.shift_right_logical(v16, 8), 8)
                + jax.lax.shift_left(v16 & 127, 1)
                + (jax.lax.shift_right_logical(v16, 7) & 1))
        return pcarry

    lax.fori_loop(0, NCHUNK, pv_body, 0)

    def issue_gather(c, b):
        pltpu.async_copy(t2_hbm.at[pv.at[c]], gbuf.at[b], gsem.at[b])

    for b in range(NG):
        issue_gather(b, b)

    def group_body(g, carry):
        for b in range(NG):
            c = g * NG + b
            o = b % NO
            pltpu.make_async_copy(
                t2_hbm.at[pv.at[c]], gbuf.at[b], gsem.at[b]).wait()

            @pl.when(c >= NO)
            def _():
                pltpu.make_async_copy(
                    obuf.at[o], out_hbm.at[pl.ds(0, CHUNK)], osem.at[o]).wait()

            poff = lax.rem(c * CHUNK, MAX_LEN)

            def row_body(i, rcarry):
                for v in range(DIM // LANES):
                    sl = pl.ds(v * LANES, LANES)
                    obuf[o, i, sl] = gbuf[b, i, sl] * SCALE + pe_v[poff + i, sl]
                return rcarry

            lax.fori_loop(0, CHUNK, row_body, 0)
            pltpu.async_copy(
                obuf.at[o], out_hbm.at[pl.ds(base + c * CHUNK, CHUNK)],
                osem.at[o])

            @pl.when(g < NCHUNK // NG - 1)
            def _():
                issue_gather(c + NG, b)
        return carry

    lax.fori_loop(0, NCHUNK // NG, group_body, 0)

    for o in range(NO):
        pltpu.make_async_copy(
            obuf.at[o], out_hbm.at[pl.ds(0, CHUNK)], osem.at[o]).wait()


def kernel(x, table):
    pe = _make_pe()
    pe = jnp.concatenate([pe, pe[:CHUNK]], axis=0)
    t2r = _fmt(jnp.swapaxes(table, 0, 1)).reshape(TROWS, DIM)
    idx = x.reshape(NW * NCHUNK, CHUNK).astype(jnp.int32)
    out = _sc_embed(idx, pe, t2r)
    return out.reshape(BATCH, SEQ, DIM)


# speedup vs baseline: 2.6069x; 2.5722x over previous
import functools
import math

import jax
import jax.numpy as jnp
from jax import lax
from jax.experimental import pallas as pl
from jax.experimental.pallas import tpu as pltpu
from jax.experimental.pallas import tpu_sc as plsc

VOCAB = 1000000
DIM = 64
MAX_LEN = 200
BATCH = 1024
SEQ = 200

NC = 2
NS = 16
NW = NC * NS

ROWS = BATCH * SEQ
ROWS_PER_W = ROWS // NW
CHUNK = 80
NCHUNK = ROWS_PER_W // CHUNK
PE_ROWS = MAX_LEN + CHUNK
NG = 8
NO = 4
LANES = 16
SCALE = math.sqrt(DIM)

TBLK = 4096
NBLK = (VOCAB + TBLK - 1) // TBLK
TROWS = NBLK * TBLK

def _fmt_body(tt_ref, t2_ref):
    blk = tt_ref[...]
    eye = jnp.eye(DIM, dtype=jnp.float32)
    t = lax.dot_general(blk, eye, (((0,), (0,)), ((), ())),
                        preferred_element_type=jnp.float32)
    lo = jnp.concatenate(
        [t[256 * k:256 * k + 128] for k in range(TBLK // 256)], axis=0)
    hi = jnp.concatenate(
        [t[256 * k + 128:256 * k + 256] for k in range(TBLK // 256)], axis=0)
    t2_ref[:, 0:DIM] = lo
    t2_ref[:, DIM:128] = hi


_fmt = pl.pallas_call(
    _fmt_body,
    grid=(NBLK,),
    in_specs=[pl.BlockSpec((DIM, TBLK), lambda i: (0, i))],
    out_specs=pl.BlockSpec((TBLK // 2, 128), lambda i: (i, 0)),
    out_shape=jax.ShapeDtypeStruct((TROWS // 2, 128), jnp.float32),
)


def _make_pe():
    w = jnp.exp(-jnp.arange(0, DIM, 2, dtype=jnp.float32) * math.log(10000.0) / DIM)
    p = jnp.arange(0, MAX_LEN, dtype=jnp.float32).reshape(MAX_LEN, 1)
    pe = jnp.zeros((MAX_LEN, DIM), dtype=jnp.float32)
    pe = pe.at[:, 0::2].set(jnp.sin(p * w))
    pe = pe.at[:, 1::2].set(jnp.cos(p * w))
    return pe


@functools.partial(
    pl.kernel,
    mesh=plsc.VectorSubcoreMesh(core_axis_name="c", subcore_axis_name="s"),
    out_type=jax.ShapeDtypeStruct((ROWS, DIM), jnp.float32),
    scratch_types=[
        pltpu.VMEM((NCHUNK, CHUNK), jnp.int32),
        pltpu.VMEM((NCHUNK, CHUNK), jnp.int32),
        pltpu.VMEM((PE_ROWS, DIM), jnp.float32),
        pltpu.VMEM((NG, CHUNK, DIM), jnp.float32),
        pltpu.VMEM((NO, CHUNK, DIM), jnp.float32),
        pltpu.SemaphoreType.DMA((NG,)),
        pltpu.SemaphoreType.DMA((NO,)),
    ],
    compiler_params=pltpu.CompilerParams(use_tc_tiling_on_sc=False),
)
def _sc_embed(idx_hbm, pe_hbm, t2_hbm, out_hbm, iv, pv, pe_v, gbuf, obuf,
              gsem, osem):
    wid = lax.axis_index("s") * NC + lax.axis_index("c")
    base = wid * ROWS_PER_W
    pltpu.sync_copy(idx_hbm.at[pl.ds(wid * NCHUNK, NCHUNK)], iv)
    pltpu.sync_copy(pe_hbm, pe_v)
    def pv_body(c, pcarry):
        for t in range(CHUNK // LANES):
            sl = pl.ds(t * LANES, LANES)
            v16 = iv[c, sl]
            pv[c, sl] = (
                jax.lax.shift_left(jax.lax.shift_right_logical(v16, 8), 8)
                + jax.lax.shift_left(v16 & 127, 1)
                + (jax.lax.shift_right_logical(v16, 7) & 1))
        return pcarry

    lax.fori_loop(0, NCHUNK, pv_body, 0)

    def issue_gather(c, b):
        pltpu.async_copy(t2_hbm.at[pv.at[c]], gbuf.at[b], gsem.at[b])

    for b in range(NG):
        issue_gather(b, b)

    def group_body(g, carry):
        for b in range(NG):
            c = g * NG + b
            o = b % NO
            pltpu.make_async_copy(
                t2_hbm.at[pv.at[c]], gbuf.at[b], gsem.at[b]).wait()

            @pl.when(c >= NO)
            def _():
                pltpu.make_async_copy(
                    obuf.at[o], out_hbm.at[pl.ds(0, CHUNK)], osem.at[o]).wait()

            poff = lax.rem(c * CHUNK, MAX_LEN)

            def row_body(i, rcarry):
                for v in range(DIM // LANES):
                    sl = pl.ds(v * LANES, LANES)
                    obuf[o, i, sl] = gbuf[b, i, sl] * SCALE + pe_v[poff + i, sl]
                return rcarry

            lax.fori_loop(0, CHUNK, row_body, 0)
            pltpu.async_copy(
                obuf.at[o], out_hbm.at[pl.ds(base + c * CHUNK, CHUNK)],
                osem.at[o])

            @pl.when(g < NCHUNK // NG - 1)
            def _():
                issue_gather(c + NG, b)
        return carry

    lax.fori_loop(0, NCHUNK // NG, group_body, 0)

    for o in range(NO):
        pltpu.make_async_copy(
            obuf.at[o], out_hbm.at[pl.ds(0, CHUNK)], osem.at[o]).wait()


def kernel(x, table):
    pe = _make_pe()
    pe = jnp.concatenate([pe, pe[:CHUNK]], axis=0)
    t2r = _fmt(jnp.swapaxes(table, 0, 1)).reshape(TROWS, DIM)
    idx = x.reshape(NW * NCHUNK, CHUNK).astype(jnp.int32)
    out = _sc_embed(idx, pe, t2r)
    return out.reshape(BATCH, SEQ, DIM)


# speedup vs baseline: 2.9324x; 1.1248x over previous
import functools
import math

import jax
import jax.numpy as jnp
from jax import lax
from jax.experimental import pallas as pl
from jax.experimental.pallas import tpu as pltpu
from jax.experimental.pallas import tpu_sc as plsc

VOCAB = 1000000
DIM = 64
MAX_LEN = 200
BATCH = 1024
SEQ = 200

NC = 2
NS = 16
NW = NC * NS

ROWS = BATCH * SEQ
ROWS_PER_W = ROWS // NW
CHUNK = 128
NCHUNK = ROWS_PER_W // CHUNK
PE_ROWS = MAX_LEN + CHUNK
NG = 5
NO = 5
LANES = 16
SCALE = math.sqrt(DIM)

TBLK = 8192
NBLK = (VOCAB + TBLK - 1) // TBLK
TROWS = NBLK * TBLK

def _fmt_body(tt_ref, t2_ref):
    blk = tt_ref[...]
    eye = jnp.eye(DIM, dtype=jnp.float32)
    t = lax.dot_general(blk, eye, (((0,), (0,)), ((), ())),
                        preferred_element_type=jnp.float32)
    lo = jnp.concatenate(
        [t[256 * k:256 * k + 128] for k in range(TBLK // 256)], axis=0)
    hi = jnp.concatenate(
        [t[256 * k + 128:256 * k + 256] for k in range(TBLK // 256)], axis=0)
    t2_ref[:, 0:DIM] = lo
    t2_ref[:, DIM:128] = hi


_fmt = pl.pallas_call(
    _fmt_body,
    grid=(NBLK,),
    in_specs=[pl.BlockSpec((DIM, TBLK), lambda i: (0, i))],
    out_specs=pl.BlockSpec((TBLK // 2, 128), lambda i: (i, 0)),
    out_shape=jax.ShapeDtypeStruct((TROWS // 2, 128), jnp.float32),
)


def _make_pe():
    w = jnp.exp(-jnp.arange(0, DIM, 2, dtype=jnp.float32) * math.log(10000.0) / DIM)
    p = jnp.arange(0, MAX_LEN, dtype=jnp.float32).reshape(MAX_LEN, 1)
    pe = jnp.zeros((MAX_LEN, DIM), dtype=jnp.float32)
    pe = pe.at[:, 0::2].set(jnp.sin(p * w))
    pe = pe.at[:, 1::2].set(jnp.cos(p * w))
    return pe


@functools.partial(
    pl.kernel,
    mesh=plsc.VectorSubcoreMesh(core_axis_name="c", subcore_axis_name="s"),
    out_type=jax.ShapeDtypeStruct((ROWS, DIM), jnp.float32),
    scratch_types=[
        pltpu.VMEM((NCHUNK, CHUNK), jnp.int32),
        pltpu.VMEM((NCHUNK, CHUNK), jnp.int32),
        pltpu.VMEM((PE_ROWS, DIM), jnp.float32),
        pltpu.VMEM((NG, CHUNK, DIM), jnp.float32),
        pltpu.VMEM((NO, CHUNK, DIM), jnp.float32),
        pltpu.SemaphoreType.DMA((NG,)),
        pltpu.SemaphoreType.DMA((NO,)),
    ],
    compiler_params=pltpu.CompilerParams(use_tc_tiling_on_sc=False),
)
def _sc_embed(idx_hbm, pe_hbm, t2_hbm, out_hbm, iv, pv, pe_v, gbuf, obuf,
              gsem, osem):
    wid = lax.axis_index("s") * NC + lax.axis_index("c")
    base = wid * ROWS_PER_W
    pltpu.sync_copy(idx_hbm.at[pl.ds(wid * NCHUNK, NCHUNK)], iv)
    pltpu.sync_copy(pe_hbm, pe_v)
    def pv_body(c, pcarry):
        for t in range(CHUNK // LANES):
            sl = pl.ds(t * LANES, LANES)
            v16 = iv[c, sl]
            pv[c, sl] = (
                jax.lax.shift_left(jax.lax.shift_right_logical(v16, 8), 8)
                + jax.lax.shift_left(v16 & 127, 1)
                + (jax.lax.shift_right_logical(v16, 7) & 1))
        return pcarry

    lax.fori_loop(0, NCHUNK, pv_body, 0)

    def issue_gather(c, b):
        pltpu.async_copy(t2_hbm.at[pv.at[c]], gbuf.at[b], gsem.at[b])

    for b in range(NG):
        issue_gather(b, b)

    def group_body(g, carry):
        for b in range(NG):
            c = g * NG + b
            o = b % NO
            pltpu.make_async_copy(
                t2_hbm.at[pv.at[c]], gbuf.at[b], gsem.at[b]).wait()

            @pl.when(c >= NO)
            def _():
                pltpu.make_async_copy(
                    obuf.at[o], out_hbm.at[pl.ds(0, CHUNK)], osem.at[o]).wait()

            poff = lax.rem(c * CHUNK, MAX_LEN)

            def row_body(i, rcarry):
                for v in range(DIM // LANES):
                    sl = pl.ds(v * LANES, LANES)
                    obuf[o, i, sl] = gbuf[b, i, sl] * SCALE + pe_v[poff + i, sl]
                return rcarry

            lax.fori_loop(0, CHUNK, row_body, 0)
            pltpu.async_copy(
                obuf.at[o], out_hbm.at[pl.ds(base + c * CHUNK, CHUNK)],
                osem.at[o])

            @pl.when(g < NCHUNK // NG - 1)
            def _():
                issue_gather(c + NG, b)
        return carry

    lax.fori_loop(0, NCHUNK // NG, group_body, 0)

    for o in range(NO):
        pltpu.make_async_copy(
            obuf.at[o], out_hbm.at[pl.ds(0, CHUNK)], osem.at[o]).wait()


def kernel(x, table):
    pe = _make_pe()
    pe = jnp.concatenate([pe, pe[:CHUNK]], axis=0)
    t2r = _fmt(jnp.swapaxes(table, 0, 1)).reshape(TROWS, DIM)
    idx = x.reshape(NW * NCHUNK, CHUNK).astype(jnp.int32)
    out = _sc_embed(idx, pe, t2r)
    return out.reshape(BATCH, SEQ, DIM)


# speedup vs baseline: 3.1121x; 1.0613x over previous
import functools
import math

import jax
import jax.numpy as jnp
from jax import lax
from jax.experimental import pallas as pl
from jax.experimental.pallas import tpu as pltpu
from jax.experimental.pallas import tpu_sc as plsc

VOCAB = 1000000
DIM = 64
MAX_LEN = 200
BATCH = 1024
SEQ = 200

NC = 2
NS = 16
NW = NC * NS

ROWS = BATCH * SEQ
ROWS_PER_W = ROWS // NW
CHUNK = 128
NCHUNK = ROWS_PER_W // CHUNK
PE_ROWS = MAX_LEN + CHUNK
NG = 5
NO = 5
LANES = 16
SCALE = math.sqrt(DIM)

TBLK = 16384
NBLK = (VOCAB + TBLK - 1) // TBLK
TROWS = NBLK * TBLK

def _fmt_body(tt_ref, t2_ref):
    blk = tt_ref[...]
    eye = jnp.eye(DIM, dtype=jnp.float32)
    t = lax.dot_general(blk, eye, (((0,), (0,)), ((), ())),
                        preferred_element_type=jnp.float32)
    lo = jnp.concatenate(
        [t[256 * k:256 * k + 128] for k in range(TBLK // 256)], axis=0)
    hi = jnp.concatenate(
        [t[256 * k + 128:256 * k + 256] for k in range(TBLK // 256)], axis=0)
    t2_ref[:, 0:DIM] = lo
    t2_ref[:, DIM:128] = hi


_fmt = pl.pallas_call(
    _fmt_body,
    grid=(NBLK,),
    in_specs=[pl.BlockSpec((DIM, TBLK), lambda i: (0, i))],
    out_specs=pl.BlockSpec((TBLK // 2, 128), lambda i: (i, 0)),
    out_shape=jax.ShapeDtypeStruct((TROWS // 2, 128), jnp.float32),
)


def _make_pe():
    w = jnp.exp(-jnp.arange(0, DIM, 2, dtype=jnp.float32) * math.log(10000.0) / DIM)
    p = jnp.arange(0, MAX_LEN, dtype=jnp.float32).reshape(MAX_LEN, 1)
    pe = jnp.zeros((MAX_LEN, DIM), dtype=jnp.float32)
    pe = pe.at[:, 0::2].set(jnp.sin(p * w))
    pe = pe.at[:, 1::2].set(jnp.cos(p * w))
    return pe


@functools.partial(
    pl.kernel,
    mesh=plsc.VectorSubcoreMesh(core_axis_name="c", subcore_axis_name="s"),
    out_type=jax.ShapeDtypeStruct((ROWS, DIM), jnp.float32),
    scratch_types=[
        pltpu.VMEM((NCHUNK, CHUNK), jnp.int32),
        pltpu.VMEM((NCHUNK, CHUNK), jnp.int32),
        pltpu.VMEM((PE_ROWS, DIM), jnp.float32),
        pltpu.VMEM((NG, CHUNK, DIM), jnp.float32),
        pltpu.VMEM((NO, CHUNK, DIM), jnp.float32),
        pltpu.SemaphoreType.DMA((NG,)),
        pltpu.SemaphoreType.DMA((NO,)),
    ],
    compiler_params=pltpu.CompilerParams(use_tc_tiling_on_sc=False),
)
def _sc_embed(idx_hbm, pe_hbm, t2_hbm, out_hbm, iv, pv, pe_v, gbuf, obuf,
              gsem, osem):
    wid = lax.axis_index("s") * NC + lax.axis_index("c")
    base = wid * ROWS_PER_W
    pltpu.sync_copy(idx_hbm.at[pl.ds(wid * NCHUNK, NCHUNK)], iv)
    pltpu.sync_copy(pe_hbm, pe_v)
    def pv_body(c, pcarry):
        for t in range(CHUNK // LANES):
            sl = pl.ds(t * LANES, LANES)
            v16 = iv[c, sl]
            pv[c, sl] = (
                jax.lax.shift_left(jax.lax.shift_right_logical(v16, 8), 8)
                + jax.lax.shift_left(v16 & 127, 1)
                + (jax.lax.shift_right_logical(v16, 7) & 1))
        return pcarry

    lax.fori_loop(0, NCHUNK, pv_body, 0)

    def issue_gather(c, b):
        pltpu.async_copy(t2_hbm.at[pv.at[c]], gbuf.at[b], gsem.at[b])

    for b in range(NG):
        issue_gather(b, b)

    def group_body(g, carry):
        for b in range(NG):
            c = g * NG + b
            o = b % NO
            pltpu.make_async_copy(
                t2_hbm.at[pv.at[c]], gbuf.at[b], gsem.at[b]).wait()

            @pl.when(c >= NO)
            def _():
                pltpu.make_async_copy(
                    obuf.at[o], out_hbm.at[pl.ds(0, CHUNK)], osem.at[o]).wait()

            poff = lax.rem(c * CHUNK, MAX_LEN)

            def row_body(i, rcarry):
                for v in range(DIM // LANES):
                    sl = pl.ds(v * LANES, LANES)
                    obuf[o, i, sl] = gbuf[b, i, sl] * SCALE + pe_v[poff + i, sl]
                return rcarry

            lax.fori_loop(0, CHUNK, row_body, 0)
            pltpu.async_copy(
                obuf.at[o], out_hbm.at[pl.ds(base + c * CHUNK, CHUNK)],
                osem.at[o])

            @pl.when(g < NCHUNK // NG - 1)
            def _():
                issue_gather(c + NG, b)
        return carry

    lax.fori_loop(0, NCHUNK // NG, group_body, 0)

    for o in range(NO):
        pltpu.make_async_copy(
            obuf.at[o], out_hbm.at[pl.ds(0, CHUNK)], osem.at[o]).wait()


def kernel(x, table):
    pe = _make_pe()
    pe = jnp.concatenate([pe, pe[:CHUNK]], axis=0)
    t2r = _fmt(jnp.swapaxes(table, 0, 1)).reshape(TROWS, DIM)
    idx = x.reshape(NW * NCHUNK, CHUNK).astype(jnp.int32)
    out = _sc_embed(idx, pe, t2r)
    return out.reshape(BATCH, SEQ, DIM)
